# baseline (speedup 1.0000x reference)
import jax
import jax.numpy as jnp
from jax import lax
from jax.experimental import pallas as pl
from jax.experimental.pallas import tpu as pltpu
from jax.experimental.pallas import tpu_sc as plsc

CLS = 1000
FEAT = 64
BATCH = 1024
CLS_P = 1024

_L = 16
_NW = 32
_BPW = BATCH // _NW
_NCHUNK = BATCH // _L
_NSEM = 4


def _shift_cmp(idx, iota, s):
    gidx = jnp.minimum(iota + s, _L - 1)
    shifted = lax.gather(
        idx, gidx[:, None],
        lax.GatherDimensionNumbers(
            offset_dims=(), collapsed_slice_dims=(0,), start_index_map=(0,)),
        slice_sizes=(1,),
        mode=lax.GatherScatterMode.PROMISE_IN_BOUNDS)
    return (shifted == idx) & (iota < _L - s)


def _sc_body(table_hbm, cls_hbm, sel_hbm, rep_hbm,
             cls_v, lp_v, gidx_v, sidx_v, rows_v, tmp_v, sem, sem2, sems):
    wid = lax.axis_index("c") * 16 + lax.axis_index("s")
    base = wid * _BPW
    iota = lax.iota(jnp.int32, _L)

    pltpu.sync_copy(cls_hbm, cls_v)

    gidx_v[pl.ds(0, _L)] = cls_v[pl.ds(base, _L)] + BATCH
    gidx_v[pl.ds(_L, _L)] = cls_v[pl.ds(base + _L, _L)] + BATCH
    sel_dma = pltpu.async_copy(table_hbm.at[gidx_v], rows_v, sem)

    def _init(j, carry):
        lp_v[pl.ds(j * _L, _L)] = jnp.full((_L,), -1, jnp.int32)
        return carry
    lax.fori_loop(0, CLS_P // _L, _init, 0)

    def _scatter_chunk(i, carry):
        idx = cls_v[pl.ds(i * _L, _L)]
        pos = iota + i * _L
        dup = iota < 0
        for s in range(1, _L):
            dup = dup | _shift_cmp(idx, iota, s)
        plsc.store_scatter(lp_v, [idx], pos, mask=jnp.logical_not(dup))
        return carry
    lax.fori_loop(0, _NCHUNK, _scatter_chunk, 0)

    def _sidx(j, carry):
        lp = lp_v[pl.ds(j * _L, _L)]
        sidx_v[pl.ds(j * _L, _L)] = jnp.where(
            lp >= 0, lp, BATCH + j * _L + iota)
        return carry
    lax.fori_loop(0, CLS_P // _L, _sidx, 0)

    pltpu.async_copy(table_hbm.at[sidx_v], tmp_v, sem2).wait()

    handles = [None] * _BPW
    for r in range(_BPW):
        if r >= _NSEM:
            handles[r - _NSEM].wait()
        handles[r] = pltpu.async_copy(
            tmp_v.at[pl.ds(0, CLS)], rep_hbm.at[base + r],
            sems.at[r % _NSEM])
    for r in range(_BPW - _NSEM, _BPW):
        handles[r].wait()

    sel_dma.wait()
    pltpu.sync_copy(rows_v, sel_hbm.at[pl.ds(base, _BPW)])


def kernel(x, cls_list, memory):
    table = jnp.concatenate(
        [x, memory, jnp.zeros((CLS_P - CLS, FEAT), memory.dtype)], axis=0)

    mesh = plsc.VectorSubcoreMesh(core_axis_name="c", subcore_axis_name="s")
    sc = pl.kernel(
        _sc_body,
        mesh=mesh,
        out_type=(
            jax.ShapeDtypeStruct((BATCH, FEAT), jnp.float32),
            jax.ShapeDtypeStruct((BATCH, CLS, FEAT), jnp.float32),
        ),
        scratch_types=[
            pltpu.VMEM((BATCH,), jnp.int32),
            pltpu.VMEM((CLS_P,), jnp.int32),
            pltpu.VMEM((_BPW,), jnp.int32),
            pltpu.VMEM((CLS_P,), jnp.int32),
            pltpu.VMEM((_BPW, FEAT), jnp.float32),
            pltpu.VMEM((CLS_P, FEAT), jnp.float32),
            pltpu.SemaphoreType.DMA,
            pltpu.SemaphoreType.DMA,
            pltpu.SemaphoreType.DMA((_NSEM,)),
        ],
        compiler_params=pltpu.CompilerParams(
            needs_layout_passes=False, use_tc_tiling_on_sc=False),
    )
    selected, tmp_rep = sc(table, cls_list)
    return selected, tmp_rep


# speedup vs baseline: 1.3050x; 1.3050x over previous
import jax
import jax.numpy as jnp
from jax import lax
from jax.experimental import pallas as pl
from jax.experimental.pallas import tpu as pltpu
from jax.experimental.pallas import tpu_sc as plsc

CLS = 1000
FEAT = 64
BATCH = 1024
CLS_P = 1024

_L = 16
_NW = 32
_CPW = CLS_P // _NW
_BPW = BATCH // _NW
_NCHUNK = BATCH // _L
_NSEM = 4


def _shift_cmp(idx, iota, s):
    gidx = jnp.minimum(iota + s, _L - 1)
    shifted = lax.gather(
        idx, gidx[:, None],
        lax.GatherDimensionNumbers(
            offset_dims=(), collapsed_slice_dims=(0,), start_index_map=(0,)),
        slice_sizes=(1,),
        mode=lax.GatherScatterMode.PROMISE_IN_BOUNDS)
    return (shifted == idx) & (iota < _L - s)


def _sc_body(table_hbm, cls_hbm, sel_hbm, tmp_hbm,
             cls_v, lp_v, gidx_v, sidx_v, rows_v, trows_v, sem, sem2):
    wid = lax.axis_index("c") * 16 + lax.axis_index("s")
    base = wid * _BPW
    cbase = wid * _CPW
    iota = lax.iota(jnp.int32, _L)

    pltpu.sync_copy(cls_hbm, cls_v)

    gidx_v[pl.ds(0, _L)] = cls_v[pl.ds(base, _L)] + BATCH
    gidx_v[pl.ds(_L, _L)] = cls_v[pl.ds(base + _L, _L)] + BATCH
    sel_dma = pltpu.async_copy(table_hbm.at[gidx_v], rows_v, sem)

    def _init(j, carry):
        lp_v[pl.ds(j * _L, _L)] = jnp.full((_L,), -1, jnp.int32)
        return carry
    lax.fori_loop(0, CLS_P // _L, _init, 0)

    def _scatter_chunk(i, carry):
        idx = cls_v[pl.ds(i * _L, _L)]
        pos = iota + i * _L
        dup = iota < 0
        for s in range(1, _L):
            dup = dup | _shift_cmp(idx, iota, s)
        plsc.store_scatter(lp_v, [idx], pos, mask=jnp.logical_not(dup))
        return carry
    lax.fori_loop(0, _NCHUNK, _scatter_chunk, 0)

    lp0 = lp_v[pl.ds(cbase, _L)]
    lp1 = lp_v[pl.ds(cbase + _L, _L)]
    sidx_v[pl.ds(0, _L)] = jnp.where(
        lp0 >= 0, lp0, cbase + BATCH + iota)
    sidx_v[pl.ds(_L, _L)] = jnp.where(
        lp1 >= 0, lp1, cbase + BATCH + _L + iota)

    pltpu.async_copy(table_hbm.at[sidx_v], trows_v, sem2).wait()

    @pl.when(wid < _NW - 1)
    def _full_block():
        pltpu.sync_copy(trows_v, tmp_hbm.at[pl.ds(cbase, _CPW)])

    @pl.when(wid == _NW - 1)
    def _tail_block():
        pltpu.sync_copy(trows_v.at[pl.ds(0, CLS - (_NW - 1) * _CPW)],
                        tmp_hbm.at[pl.ds(cbase, CLS - (_NW - 1) * _CPW)])

    sel_dma.wait()
    pltpu.sync_copy(rows_v, sel_hbm.at[pl.ds(base, _BPW)])


def _bc_body(tmp_hbm, rep_hbm, tmp_v, sems):
    wid = lax.axis_index("c") * 16 + lax.axis_index("s")
    base = wid * _BPW

    pltpu.sync_copy(tmp_hbm, tmp_v)

    handles = [None] * _BPW
    for r in range(_BPW):
        if r >= _NSEM:
            handles[r - _NSEM].wait()
        handles[r] = pltpu.async_copy(
            tmp_v, rep_hbm.at[base + r], sems.at[r % _NSEM])
    for r in range(_BPW - _NSEM, _BPW):
        handles[r].wait()


def kernel(x, cls_list, memory):
    table = jnp.concatenate(
        [x, memory, jnp.zeros((CLS_P - CLS, FEAT), memory.dtype)], axis=0)

    mesh = plsc.VectorSubcoreMesh(core_axis_name="c", subcore_axis_name="s")
    sc = pl.kernel(
        _sc_body,
        mesh=mesh,
        out_type=(
            jax.ShapeDtypeStruct((BATCH, FEAT), jnp.float32),
            jax.ShapeDtypeStruct((CLS, FEAT), jnp.float32),
        ),
        scratch_types=[
            pltpu.VMEM((BATCH,), jnp.int32),
            pltpu.VMEM((CLS_P,), jnp.int32),
            pltpu.VMEM((_BPW,), jnp.int32),
            pltpu.VMEM((_CPW,), jnp.int32),
            pltpu.VMEM((_BPW, FEAT), jnp.float32),
            pltpu.VMEM((_CPW, FEAT), jnp.float32),
            pltpu.SemaphoreType.DMA,
            pltpu.SemaphoreType.DMA,
        ],
        compiler_params=pltpu.CompilerParams(
            needs_layout_passes=False, use_tc_tiling_on_sc=False),
    )
    selected, tmp = sc(table, cls_list)

    bc = pl.kernel(
        _bc_body,
        mesh=plsc.VectorSubcoreMesh(core_axis_name="c", subcore_axis_name="s"),
        out_type=jax.ShapeDtypeStruct((BATCH, CLS, FEAT), jnp.float32),
        scratch_types=[
            pltpu.VMEM((CLS, FEAT), jnp.float32),
            pltpu.SemaphoreType.DMA((_NSEM,)),
        ],
        compiler_params=pltpu.CompilerParams(
            needs_layout_passes=False, use_tc_tiling_on_sc=True),
    )
    tmp_rep = bc(tmp)

    return selected, tmp_rep
